# Initial kernel scaffold; baseline (speedup 1.0000x reference)
import jax
import jax.numpy as jnp
from jax.experimental import pallas as pl


def kernel(x, edge_index, W_self1, W_neigh1, b1, W_self2, W_neigh2, b2, W_self3, W_neigh3, b3):
    raise NotImplementedError("write your pallas kernel here")


# speedup vs baseline: 8.7913x; 8.7913x over previous
import functools

import jax
import jax.numpy as jnp
from jax import lax
from jax.experimental import pallas as pl
from jax.experimental.pallas import tpu as pltpu
from jax.experimental.pallas import tpu_sc as plsc

_NC = 2
_NS = 16
_B = 80


def _make_sc_segsum(N, D, E):
    NW = _NC * _NS
    assert E % NW == 0
    epw = E // NW
    assert epw % _B == 0
    steps = epw // _B
    rows_pt = (N // _NS) // 8 * 8
    tail = N - _NS * rows_pt
    assert tail % 8 == 0

    mesh = plsc.VectorSubcoreMesh(
        core_axis_name="c", subcore_axis_name="s",
        num_cores=_NC, num_subcores=_NS)

    out_type = jax.ShapeDtypeStruct((_NC, N, D), jnp.float32)
    scratch = [
        pltpu.VMEM((_B, D), jnp.float32),
        pltpu.VMEM_SHARED((N, D), jnp.float32),
        pltpu.SemaphoreType.DMA,
        pltpu.VMEM((_B, D), jnp.float32),
        pltpu.SemaphoreType.DMA,
        pltpu.VMEM((_B,), jnp.int32),
        pltpu.VMEM((_B,), jnp.int32),
        pltpu.VMEM((_B,), jnp.int32),
        pltpu.VMEM((_B,), jnp.int32),
        pltpu.SemaphoreType.DMA,
        pltpu.SemaphoreType.DMA,
    ]

    def body(*refs):
        (table, srcv, dstv, zeros_d,
         aggp, rows0, agg_s, sem0,
         rows1, sem1, isrc0, idst0, isrc1, idst1, semi0, semi1) = refs
        c = lax.axis_index("c")
        s = lax.axis_index("s")
        wid = s * _NC + c
        r0 = s * rows_pt

        pltpu.sync_copy(zeros_d.at[pl.ds(r0, rows_pt)], agg_s.at[pl.ds(r0, rows_pt)])
        if tail:
            t0 = _NS * rows_pt

            @pl.when(s == 0)
            def _zero_tail():
                pltpu.sync_copy(zeros_d.at[pl.ds(t0, tail)], agg_s.at[pl.ds(t0, tail)])
        plsc.subcore_barrier()

        def load_idx(j, isrc, idst, semi):
            e0 = wid * epw + jnp.minimum(j, steps - 1) * _B
            pltpu.async_copy(srcv.at[pl.ds(e0, _B)], isrc, semi)
            pltpu.async_copy(dstv.at[pl.ds(e0, _B)], idst, semi)

        def wait_idx(isrc, idst, semi):
            pltpu.make_async_copy(srcv.at[pl.ds(0, _B)], isrc, semi).wait()
            pltpu.make_async_copy(dstv.at[pl.ds(0, _B)], idst, semi).wait()

        def gather(isrc, rows, sem):
            pltpu.async_copy(table.at[isrc], rows, sem)

        def wait_rows(rows, sem):
            pltpu.make_async_copy(table.at[isrc0], rows, sem).wait()

        def scatter(rows, idst):
            pltpu.sync_copy(rows, agg_s.at[idst], add=True)

        assert steps % 2 == 1
        load_idx(0, isrc0, idst0, semi0)
        wait_idx(isrc0, idst0, semi0)
        gather(isrc0, rows0, sem0)
        load_idx(1, isrc1, idst1, semi1)

        def pair(jj, carry):
            j1 = 2 * jj + 1
            wait_idx(isrc1, idst1, semi1)
            gather(isrc1, rows1, sem1)
            wait_rows(rows0, sem0)
            scatter(rows0, idst0)
            load_idx(j1 + 1, isrc0, idst0, semi0)
            wait_idx(isrc0, idst0, semi0)
            gather(isrc0, rows0, sem0)
            wait_rows(rows1, sem1)
            scatter(rows1, idst1)
            load_idx(j1 + 2, isrc1, idst1, semi1)
            return carry

        lax.fori_loop(0, steps // 2, pair, 0)
        wait_idx(isrc1, idst1, semi1)
        wait_rows(rows0, sem0)
        scatter(rows0, idst0)
        plsc.subcore_barrier()

        pltpu.sync_copy(agg_s.at[pl.ds(r0, rows_pt)], aggp.at[c, pl.ds(r0, rows_pt)])
        if tail:
            t0 = _NS * rows_pt

            @pl.when(s == 0)
            def _write_tail():
                pltpu.sync_copy(agg_s.at[pl.ds(t0, tail)], aggp.at[c, pl.ds(t0, tail)])

    return pl.kernel(body, out_type=out_type, mesh=mesh, scratch_types=scratch)


def _make_sc_deg(N, E):
    NW = _NC * _NS
    epw = E // NW
    steps = epw // _B
    rows_pt = (N // _NS) // 8 * 8
    tail = N - _NS * rows_pt
    mesh = plsc.VectorSubcoreMesh(
        core_axis_name="c", subcore_axis_name="s",
        num_cores=_NC, num_subcores=_NS)
    scratch = [
        pltpu.VMEM((_B, 128), jnp.float32),
        pltpu.VMEM_SHARED((N, 128), jnp.float32),
        pltpu.VMEM((_B,), jnp.int32),
        pltpu.VMEM((_B,), jnp.int32),
        pltpu.SemaphoreType.DMA,
        pltpu.SemaphoreType.DMA,
    ]

    def body(dstv, zeros_d, ones_h, degp, ones_v, deg_s, idst0, idst1, semi0, semi1):
        c = lax.axis_index("c")
        s = lax.axis_index("s")
        wid = s * _NC + c
        r0 = s * rows_pt
        pltpu.sync_copy(zeros_d.at[pl.ds(r0, rows_pt)], deg_s.at[pl.ds(r0, rows_pt)])
        pltpu.sync_copy(ones_h, ones_v)
        if tail:
            t0 = _NS * rows_pt

            @pl.when(s == 0)
            def _zero_tail():
                pltpu.sync_copy(zeros_d.at[pl.ds(t0, tail)], deg_s.at[pl.ds(t0, tail)])
        plsc.subcore_barrier()

        def load_idx(j, idst, semi):
            e0 = wid * epw + jnp.minimum(j, steps - 1) * _B
            pltpu.async_copy(dstv.at[pl.ds(e0, _B)], idst, semi)

        def wait_idx(idst, semi):
            pltpu.make_async_copy(dstv.at[pl.ds(0, _B)], idst, semi).wait()

        def scatter(idst):
            pltpu.sync_copy(ones_v, deg_s.at[idst], add=True)

        assert steps % 2 == 1
        load_idx(0, idst0, semi0)
        load_idx(1, idst1, semi1)

        def pair(jj, carry):
            j1 = 2 * jj + 1
            wait_idx(idst0, semi0)
            scatter(idst0)
            load_idx(j1 + 1, idst0, semi0)
            wait_idx(idst1, semi1)
            scatter(idst1)
            load_idx(j1 + 2, idst1, semi1)
            return carry

        lax.fori_loop(0, steps // 2, pair, 0)
        wait_idx(idst0, semi0)
        scatter(idst0)
        wait_idx(idst1, semi1)
        plsc.subcore_barrier()

        pltpu.sync_copy(deg_s.at[pl.ds(r0, rows_pt)], degp.at[c, pl.ds(r0, rows_pt)])
        if tail:
            t0 = _NS * rows_pt

            @pl.when(s == 0)
            def _write_tail():
                pltpu.sync_copy(deg_s.at[pl.ds(t0, tail)], degp.at[c, pl.ds(t0, tail)])

    return pl.kernel(body, out_type=jax.ShapeDtypeStruct((_NC, N, 128), jnp.float32),
                     mesh=mesh, scratch_types=scratch)


def _inv_deg(degp):
    deg = degp[0, :, 0:1] + degp[1, :, 0:1]
    return 1.0 / jnp.maximum(deg, 1.0)


def _tc1_body(x, aggp, degp, ws, wn, b, h1a, h1b):
    inv = _inv_deg(degp[...])
    agg = (aggp[0] + aggp[1]) * inv
    h = (jnp.dot(x[...], ws[...], preferred_element_type=jnp.float32)
         + jnp.dot(agg, wn[...], preferred_element_type=jnp.float32)
         + b[...])
    h = jnp.maximum(h, 0.0)
    h1a[...] = h[:, :128]
    h1b[...] = h[:, 128:]


def _tc2_body(h1a, h1b, a2ap, a2bp, degp, ws2, wn2, b2, wn3, ws3, b3, p, q):
    inv = _inv_deg(degp[...])
    a2a = (a2ap[0] + a2ap[1]) * inv
    a2b = (a2bp[0] + a2bp[1]) * inv
    w_s, w_n = ws2[...], wn2[...]
    h2 = (jnp.dot(h1a[...], w_s[:128], preferred_element_type=jnp.float32)
          + jnp.dot(h1b[...], w_s[128:], preferred_element_type=jnp.float32)
          + jnp.dot(a2a, w_n[:128], preferred_element_type=jnp.float32)
          + jnp.dot(a2b, w_n[128:], preferred_element_type=jnp.float32)
          + b2[...])
    h2 = jnp.maximum(h2, 0.0)
    p[...] = jnp.dot(h2, wn3[...], preferred_element_type=jnp.float32)
    q[...] = jnp.dot(h2, ws3[...], preferred_element_type=jnp.float32) + b3[...]


def _tc3_body(qq, aggp, degp, out):
    inv = _inv_deg(degp[...])
    out[...] = qq[...] + (aggp[0] + aggp[1]) * inv


def _row_block_call(body, n_out, N, R, in_specs_shapes, out_dim):
    grid = (N // R,)
    in_specs = []
    for shp in in_specs_shapes:
        if shp[0] == "rows":
            d = shp[1]
            in_specs.append(pl.BlockSpec((R, d), lambda i: (i, 0)))
        elif shp[0] == "parts":
            d = shp[1]
            in_specs.append(pl.BlockSpec((2, R, d), lambda i: (0, i, 0)))
        else:
            dims = shp[1]
            in_specs.append(pl.BlockSpec(dims, lambda i, n=len(dims): (0,) * n))
    out_specs = [pl.BlockSpec((R, d), lambda i: (i, 0)) for d in out_dim]
    out_shape = [jax.ShapeDtypeStruct((N, d), jnp.float32) for d in out_dim]
    if n_out == 1:
        out_specs, out_shape = out_specs[0], out_shape[0]
    return pl.pallas_call(body, grid=grid, in_specs=in_specs,
                          out_specs=out_specs, out_shape=out_shape)


def kernel(x, edge_index, W_self1, W_neigh1, b1,
           W_self2, W_neigh2, b2, W_self3, W_neigh3, b3):
    N, DIN = x.shape
    E = edge_index.shape[1]
    DH = W_self1.shape[1]
    DOUT = W_self3.shape[1]
    R = 1000

    src = edge_index[0]
    dst = edge_index[1]
    zeros_d = jnp.zeros((N, 128), jnp.float32)
    ones_h = jnp.ones((_B, 128), jnp.float32)
    b1r = b1.reshape(1, DH)
    b2r = b2.reshape(1, DH)
    b3r = b3.reshape(1, DOUT)

    segsum = _make_sc_segsum(N, 128, E)
    sc_deg = _make_sc_deg(N, E)

    degp = sc_deg(dst, zeros_d, ones_h)
    aggp1 = segsum(x, src, dst, zeros_d)
    tc1 = _row_block_call(
        _tc1_body, 2, N, R,
        [("rows", DIN), ("parts", 128), ("parts", 128),
         ("full", (DIN, DH)), ("full", (DIN, DH)), ("full", (1, DH))],
        [128, 128])
    h1a, h1b = tc1(x, aggp1, degp, W_self1, W_neigh1, b1r)

    a2ap = segsum(h1a, src, dst, zeros_d)
    a2bp = segsum(h1b, src, dst, zeros_d)
    tc2 = _row_block_call(
        _tc2_body, 2, N, R,
        [("rows", 128), ("rows", 128), ("parts", 128), ("parts", 128),
         ("parts", 128), ("full", (DH, DH)), ("full", (DH, DH)),
         ("full", (1, DH)), ("full", (DH, DOUT)), ("full", (DH, DOUT)),
         ("full", (1, DOUT))],
        [DOUT, DOUT])
    p, q = tc2(h1a, h1b, a2ap, a2bp, degp, W_self2, W_neigh2, b2r,
               W_neigh3, W_self3, b3r)

    aggp3 = segsum(p, src, dst, zeros_d)
    tc3 = _row_block_call(
        _tc3_body, 1, N, R,
        [("rows", DOUT), ("parts", DOUT), ("parts", 128)],
        [DOUT])
    return tc3(q, aggp3, degp)
